# baseline (speedup 1.0000x reference)
import functools

import jax
import jax.numpy as jnp
from jax import lax
from jax.experimental import pallas as pl
from jax.experimental.pallas import tpu as pltpu
from jax.experimental.pallas import tpu_sc as plsc

BATCH = 1024
SEQ = 81
VOCAB = 11
GRID = 9
H = 512
VPAD = 16
EPS = 1e-5

TOKENS = BATCH * SEQ
NW = 32
ROWS_PER_W = TOKENS // NW
CHUNK = 96
NCHUNK = ROWS_PER_W // CHUNK


def _table_body(vals_ref, vw_ref, rw_ref, cw_ref, bw_ref, bias_ref, g_ref,
                b_ref, out_ref, idx_ref):
    n = SEQ * VPAD
    r = lax.broadcasted_iota(jnp.int32, (n, 1), 0)
    s_idx = r // VPAD
    v_idx = r % VPAD
    row_ids = s_idx // GRID
    col_ids = s_idx % GRID
    box_ids = (row_ids // 3) * 3 + (col_ids // 3)

    oh_v = (v_idx == lax.broadcasted_iota(jnp.int32, (n, VOCAB), 1)).astype(jnp.float32)
    oh_r = (row_ids == lax.broadcasted_iota(jnp.int32, (n, GRID), 1)).astype(jnp.float32)
    oh_c = (col_ids == lax.broadcasted_iota(jnp.int32, (n, GRID), 1)).astype(jnp.float32)
    oh_b = (box_ids == lax.broadcasted_iota(jnp.int32, (n, GRID), 1)).astype(jnp.float32)

    x = jnp.dot(oh_v, vw_ref[...], preferred_element_type=jnp.float32)
    x = x + jnp.dot(oh_r, rw_ref[...], preferred_element_type=jnp.float32)
    x = x + jnp.dot(oh_c, cw_ref[...], preferred_element_type=jnp.float32)
    x = x + jnp.dot(oh_b, bw_ref[...], preferred_element_type=jnp.float32)
    x = x + bias_ref[...]

    mean = jnp.mean(x, axis=1, keepdims=True)
    xc = x - mean
    var = jnp.mean(xc * xc, axis=1, keepdims=True)
    y = xc / jnp.sqrt(var + EPS) * g_ref[...] + b_ref[...]
    out_ref[...] = y

    s_tok = lax.broadcasted_iota(jnp.int32, (BATCH, SEQ), 1)
    idx_ref[...] = vals_ref[...] + s_tok * VPAD


def _build_table(values, vw, rw, cw, bw, bias, gamma, beta):
    return pl.pallas_call(
        _table_body,
        out_shape=(
            jax.ShapeDtypeStruct((SEQ * VPAD, H), jnp.float32),
            jax.ShapeDtypeStruct((BATCH, SEQ), jnp.int32),
        ),
    )(values, vw, rw, cw, bw, bias.reshape(1, H), gamma.reshape(1, H),
      beta.reshape(1, H))


def _sc_gather_body(t_hbm, idx_hbm, out_hbm, idx_s, buf0, buf1, sem0, sem1):
    wid = lax.axis_index("s") * 2 + lax.axis_index("c")
    base = wid * ROWS_PER_W
    pltpu.sync_copy(idx_hbm.at[wid], idx_s)

    bufs = (buf0, buf1)
    sems = (sem0, sem1)

    def start(c):
        return pltpu.async_copy(t_hbm.at[idx_s.at[c]], bufs[c % 2], sems[c % 2])

    cp = start(0)
    for c in range(NCHUNK):
        nxt = start(c + 1) if c + 1 < NCHUNK else None
        cp.wait()
        pltpu.sync_copy(bufs[c % 2], out_hbm.at[pl.ds(base + c * CHUNK, CHUNK)])
        cp = nxt


def _sc_gather(t2d, idx):
    idx3 = idx.reshape(NW, NCHUNK, CHUNK)
    mesh = plsc.VectorSubcoreMesh(core_axis_name="c", subcore_axis_name="s")
    k = functools.partial(
        pl.kernel,
        mesh=mesh,
        out_type=jax.ShapeDtypeStruct((TOKENS, H), jnp.float32),
        scratch_types=[
            pltpu.VMEM((NCHUNK, CHUNK), jnp.int32),
            pltpu.VMEM((CHUNK, H), jnp.float32),
            pltpu.VMEM((CHUNK, H), jnp.float32),
            pltpu.SemaphoreType.DMA,
            pltpu.SemaphoreType.DMA,
        ],
    )(_sc_gather_body)
    return k(t2d, idx3)


def kernel(values, value_embed_w, row_embed_w, col_embed_w, box_embed_w, input_bias, ln_gamma, ln_beta):
    t2d, idx = _build_table(values.astype(jnp.int32), value_embed_w,
                            row_embed_w, col_embed_w, box_embed_w,
                            input_bias, ln_gamma, ln_beta)
    out = _sc_gather(t2d, idx)
    return out.reshape(BATCH, SEQ, H)


# speedup vs baseline: 1.1314x; 1.1314x over previous
import functools

import jax
import jax.numpy as jnp
from jax import lax
from jax.experimental import pallas as pl
from jax.experimental.pallas import tpu as pltpu
from jax.experimental.pallas import tpu_sc as plsc

BATCH = 1024
SEQ = 81
VOCAB = 11
GRID = 9
H = 512
VPAD = 16
EPS = 1e-5

TOKENS = BATCH * SEQ
NW = 32
BATCH_PER_W = BATCH // NW


def _table_body(vals_ref, vw_ref, rw_ref, cw_ref, bw_ref, bias_ref, g_ref,
                b_ref, out_ref, idx_ref):
    n = SEQ * VPAD
    r = lax.broadcasted_iota(jnp.int32, (n, 1), 0)
    s_idx = r // VPAD
    v_idx = r % VPAD
    row_ids = s_idx // GRID
    col_ids = s_idx % GRID
    box_ids = (row_ids // 3) * 3 + (col_ids // 3)

    oh_v = (v_idx == lax.broadcasted_iota(jnp.int32, (n, VOCAB), 1)).astype(jnp.float32)
    oh_r = (row_ids == lax.broadcasted_iota(jnp.int32, (n, GRID), 1)).astype(jnp.float32)
    oh_c = (col_ids == lax.broadcasted_iota(jnp.int32, (n, GRID), 1)).astype(jnp.float32)
    oh_b = (box_ids == lax.broadcasted_iota(jnp.int32, (n, GRID), 1)).astype(jnp.float32)

    x = jnp.dot(oh_v, vw_ref[...], preferred_element_type=jnp.float32)
    x = x + jnp.dot(oh_r, rw_ref[...], preferred_element_type=jnp.float32)
    x = x + jnp.dot(oh_c, cw_ref[...], preferred_element_type=jnp.float32)
    x = x + jnp.dot(oh_b, bw_ref[...], preferred_element_type=jnp.float32)
    x = x + bias_ref[...]

    mean = jnp.mean(x, axis=1, keepdims=True)
    xc = x - mean
    var = jnp.mean(xc * xc, axis=1, keepdims=True)
    y = xc / jnp.sqrt(var + EPS) * g_ref[...] + b_ref[...]
    out_ref[...] = y

    s_tok = lax.broadcasted_iota(jnp.int32, (BATCH, SEQ), 1)
    idx_ref[...] = vals_ref[...] + s_tok * VPAD


def _build_table(values, vw, rw, cw, bw, bias, gamma, beta):
    return pl.pallas_call(
        _table_body,
        out_shape=(
            jax.ShapeDtypeStruct((SEQ * VPAD, H), jnp.float32),
            jax.ShapeDtypeStruct((BATCH, SEQ), jnp.int32),
        ),
    )(values, vw, rw, cw, bw, bias.reshape(1, H), gamma.reshape(1, H),
      beta.reshape(1, H))


def _sc_gather_body(t_hbm, idx_hbm, out_hbm, idx_s, buf0, buf1, row0, row1,
                    sem0, sem1):
    wid = lax.axis_index("s") * 2 + lax.axis_index("c")
    base = wid * BATCH_PER_W
    pltpu.sync_copy(idx_hbm.at[wid], idx_s)

    def start_gather(c, buf, row, sem):
        pltpu.async_copy(t_hbm.at[idx_s.at[c, pl.ds(0, 80)]],
                         buf.at[pl.ds(0, 80)], sem)
        pltpu.async_copy(t_hbm.at[idx_s.at[c, pl.ds(80, 8)]], row, sem)

    def wait_gather(buf, row, sem):
        pltpu.make_async_copy(t_hbm.at[idx_s.at[0, pl.ds(0, 80)]],
                              buf.at[pl.ds(0, 80)], sem).wait()
        pltpu.make_async_copy(t_hbm.at[idx_s.at[0, pl.ds(80, 8)]], row,
                              sem).wait()

    start_gather(0, buf0, row0, sem0)

    bufs = (buf0, buf1)
    rows = (row0, row1)
    sems = (sem0, sem1)
    for c in range(BATCH_PER_W):
        b = c % 2
        if c + 1 < BATCH_PER_W:
            start_gather(c + 1, bufs[1 - b], rows[1 - b], sems[1 - b])
        wait_gather(bufs[b], rows[b], sems[b])
        for j in range(H // 16):
            bufs[b][80, pl.ds(j * 16, 16)] = rows[b][0, pl.ds(j * 16, 16)]
        pltpu.sync_copy(bufs[b], out_hbm.at[base + c])


SEQ_PAD = 88


def _sc_gather(t2d, idx):
    idx3 = jnp.pad(idx, ((0, 0), (0, SEQ_PAD - SEQ)), mode='edge').reshape(NW, BATCH_PER_W, SEQ_PAD)
    mesh = plsc.VectorSubcoreMesh(core_axis_name="c", subcore_axis_name="s")
    k = functools.partial(
        pl.kernel,
        mesh=mesh,
        out_type=jax.ShapeDtypeStruct((BATCH, SEQ, H), jnp.float32),
        scratch_types=[
            pltpu.VMEM((BATCH_PER_W, SEQ_PAD), jnp.int32),
            pltpu.VMEM((SEQ, H), jnp.float32),
            pltpu.VMEM((SEQ, H), jnp.float32),
            pltpu.VMEM((8, H), jnp.float32),
            pltpu.VMEM((8, H), jnp.float32),
            pltpu.SemaphoreType.DMA,
            pltpu.SemaphoreType.DMA,
        ],
    )(_sc_gather_body)
    return k(t2d, idx3)


def kernel(values, value_embed_w, row_embed_w, col_embed_w, box_embed_w, input_bias, ln_gamma, ln_beta):
    t2d, idx = _build_table(values.astype(jnp.int32), value_embed_w,
                            row_embed_w, col_embed_w, box_embed_w,
                            input_bias, ln_gamma, ln_beta)
    return _sc_gather(t2d, idx)


# speedup vs baseline: 1.1634x; 1.0282x over previous
import functools

import jax
import jax.numpy as jnp
from jax import lax
from jax.experimental import pallas as pl
from jax.experimental.pallas import tpu as pltpu
from jax.experimental.pallas import tpu_sc as plsc

BATCH = 1024
SEQ = 81
VOCAB = 11
GRID = 9
H = 512
VPAD = 16
EPS = 1e-5

TOKENS = BATCH * SEQ
NW = 32
BATCH_PER_W = BATCH // NW


def _table_body(vals_ref, vw_ref, rw_ref, cw_ref, bw_ref, bias_ref, g_ref,
                b_ref, out_ref, idx_ref):
    n = SEQ * VPAD
    r = lax.broadcasted_iota(jnp.int32, (n, 1), 0)
    s_idx = r // VPAD
    v_idx = r % VPAD
    row_ids = s_idx // GRID
    col_ids = s_idx % GRID
    box_ids = (row_ids // 3) * 3 + (col_ids // 3)

    oh_v = (v_idx == lax.broadcasted_iota(jnp.int32, (n, VOCAB), 1)).astype(jnp.float32)
    oh_r = (row_ids == lax.broadcasted_iota(jnp.int32, (n, GRID), 1)).astype(jnp.float32)
    oh_c = (col_ids == lax.broadcasted_iota(jnp.int32, (n, GRID), 1)).astype(jnp.float32)
    oh_b = (box_ids == lax.broadcasted_iota(jnp.int32, (n, GRID), 1)).astype(jnp.float32)

    x = jnp.dot(oh_v, vw_ref[...], preferred_element_type=jnp.float32)
    x = x + jnp.dot(oh_r, rw_ref[...], preferred_element_type=jnp.float32)
    x = x + jnp.dot(oh_c, cw_ref[...], preferred_element_type=jnp.float32)
    x = x + jnp.dot(oh_b, bw_ref[...], preferred_element_type=jnp.float32)
    x = x + bias_ref[...]

    mean = jnp.mean(x, axis=1, keepdims=True)
    xc = x - mean
    var = jnp.mean(xc * xc, axis=1, keepdims=True)
    y = xc / jnp.sqrt(var + EPS) * g_ref[...] + b_ref[...]
    out_ref[...] = y

    s_tok = lax.broadcasted_iota(jnp.int32, (BATCH, SEQ), 1)
    idx_ref[...] = vals_ref[...] + s_tok * VPAD


def _build_table(values, vw, rw, cw, bw, bias, gamma, beta):
    return pl.pallas_call(
        _table_body,
        out_shape=(
            jax.ShapeDtypeStruct((SEQ * VPAD, H), jnp.float32),
            jax.ShapeDtypeStruct((BATCH, SEQ), jnp.int32),
        ),
    )(values, vw, rw, cw, bw, bias.reshape(1, H), gamma.reshape(1, H),
      beta.reshape(1, H))


def _sc_gather_body(t_hbm, idx_hbm, out_hbm, idx_s, buf0, buf1, row0, row1,
                    sem0, sem1, ssem0, ssem1):
    wid = lax.axis_index("s") * 2 + lax.axis_index("c")
    base = wid * BATCH_PER_W
    pltpu.sync_copy(idx_hbm.at[wid], idx_s)

    def start_gather(c, buf, row, sem):
        pltpu.async_copy(t_hbm.at[idx_s.at[c, pl.ds(0, 80)]],
                         buf.at[pl.ds(0, 80)], sem)
        pltpu.async_copy(t_hbm.at[idx_s.at[c, pl.ds(80, 8)]], row, sem)

    def wait_gather(buf, row, sem):
        pltpu.make_async_copy(t_hbm.at[idx_s.at[0, pl.ds(0, 80)]],
                              buf.at[pl.ds(0, 80)], sem).wait()
        pltpu.make_async_copy(t_hbm.at[idx_s.at[0, pl.ds(80, 8)]], row,
                              sem).wait()

    def wait_scatter(c, buf, sem):
        pltpu.make_async_copy(buf, out_hbm.at[base + c], sem).wait()

    start_gather(0, buf0, row0, sem0)

    bufs = (buf0, buf1)
    rows = (row0, row1)
    sems = (sem0, sem1)
    ssems = (ssem0, ssem1)
    for c in range(BATCH_PER_W):
        b = c % 2
        if c + 1 < BATCH_PER_W:
            if c >= 1:
                wait_scatter(c - 1, bufs[1 - b], ssems[1 - b])
            start_gather(c + 1, bufs[1 - b], rows[1 - b], sems[1 - b])
        wait_gather(bufs[b], rows[b], sems[b])
        for j in range(H // 16):
            bufs[b][80, pl.ds(j * 16, 16)] = rows[b][0, pl.ds(j * 16, 16)]
        pltpu.async_copy(bufs[b], out_hbm.at[base + c], ssems[b])
    wait_scatter(BATCH_PER_W - 2, bufs[0], ssems[0])
    wait_scatter(BATCH_PER_W - 1, bufs[1], ssems[1])


SEQ_PAD = 88


def _sc_gather(t2d, idx):
    idx3 = jnp.pad(idx, ((0, 0), (0, SEQ_PAD - SEQ)), mode='edge').reshape(NW, BATCH_PER_W, SEQ_PAD)
    mesh = plsc.VectorSubcoreMesh(core_axis_name="c", subcore_axis_name="s")
    k = functools.partial(
        pl.kernel,
        mesh=mesh,
        out_type=jax.ShapeDtypeStruct((BATCH, SEQ, H), jnp.float32),
        scratch_types=[
            pltpu.VMEM((BATCH_PER_W, SEQ_PAD), jnp.int32),
            pltpu.VMEM((SEQ, H), jnp.float32),
            pltpu.VMEM((SEQ, H), jnp.float32),
            pltpu.VMEM((8, H), jnp.float32),
            pltpu.VMEM((8, H), jnp.float32),
            pltpu.SemaphoreType.DMA,
            pltpu.SemaphoreType.DMA,
            pltpu.SemaphoreType.DMA,
            pltpu.SemaphoreType.DMA,
        ],
    )(_sc_gather_body)
    return k(t2d, idx3)


def kernel(values, value_embed_w, row_embed_w, col_embed_w, box_embed_w, input_bias, ln_gamma, ln_beta):
    t2d, idx = _build_table(values.astype(jnp.int32), value_embed_w,
                            row_embed_w, col_embed_w, box_embed_w,
                            input_bias, ln_gamma, ln_beta)
    return _sc_gather(t2d, idx)


# speedup vs baseline: 2.0290x; 1.7440x over previous
import functools

import jax
import jax.numpy as jnp
from jax import lax
from jax.experimental import pallas as pl
from jax.experimental.pallas import tpu as pltpu

BATCH = 1024
SEQ = 81
VOCAB = 11
GRID = 9
H = 512
VPAD = 16
EPS = 1e-5

BB = 64


def _table_body(vw_ref, rw_ref, cw_ref, bw_ref, bias_ref, g_ref, b_ref, out_ref):
    n = SEQ * VPAD
    r = lax.broadcasted_iota(jnp.int32, (n, 1), 0)
    s_idx = r // VPAD
    v_idx = r % VPAD
    row_ids = s_idx // GRID
    col_ids = s_idx % GRID
    box_ids = (row_ids // 3) * 3 + (col_ids // 3)

    oh_v = (v_idx == lax.broadcasted_iota(jnp.int32, (n, VOCAB), 1)).astype(jnp.float32)
    oh_r = (row_ids == lax.broadcasted_iota(jnp.int32, (n, GRID), 1)).astype(jnp.float32)
    oh_c = (col_ids == lax.broadcasted_iota(jnp.int32, (n, GRID), 1)).astype(jnp.float32)
    oh_b = (box_ids == lax.broadcasted_iota(jnp.int32, (n, GRID), 1)).astype(jnp.float32)

    x = jnp.dot(oh_v, vw_ref[...], preferred_element_type=jnp.float32)
    x = x + jnp.dot(oh_r, rw_ref[...], preferred_element_type=jnp.float32)
    x = x + jnp.dot(oh_c, cw_ref[...], preferred_element_type=jnp.float32)
    x = x + jnp.dot(oh_b, bw_ref[...], preferred_element_type=jnp.float32)
    x = x + bias_ref[...]

    mean = jnp.mean(x, axis=1, keepdims=True)
    xc = x - mean
    var = jnp.mean(xc * xc, axis=1, keepdims=True)
    y = xc / jnp.sqrt(var + EPS) * g_ref[...] + b_ref[...]
    out_ref[...] = y


def _build_table(vw, rw, cw, bw, bias, gamma, beta):
    return pl.pallas_call(
        _table_body,
        out_shape=jax.ShapeDtypeStruct((SEQ * VPAD, H), jnp.float32),
    )(vw, rw, cw, bw, bias.reshape(1, H), gamma.reshape(1, H), beta.reshape(1, H))


def _gather_body(values_ref, t_ref, out_ref):
    for s in range(SEQ):
        vcol = values_ref[:, s : s + 1]
        oh = (vcol == lax.broadcasted_iota(jnp.int32, (BB, VPAD), 1)).astype(jnp.float32)
        ts = t_ref[s]
        out_ref[:, s, :] = jnp.dot(oh, ts, preferred_element_type=jnp.float32)


def _gather(values, table3):
    grid = BATCH // BB
    return pl.pallas_call(
        _gather_body,
        grid=(grid,),
        in_specs=[
            pl.BlockSpec((BB, SEQ), lambda i: (i, 0)),
            pl.BlockSpec((SEQ, VPAD, H), lambda i: (0, 0, 0)),
        ],
        out_specs=pl.BlockSpec((BB, SEQ, H), lambda i: (i, 0, 0)),
        out_shape=jax.ShapeDtypeStruct((BATCH, SEQ, H), jnp.float32),
        compiler_params=pltpu.CompilerParams(
            dimension_semantics=("parallel",),
        ),
    )(values, table3)


def kernel(values, value_embed_w, row_embed_w, col_embed_w, box_embed_w, input_bias, ln_gamma, ln_beta):
    t2d = _build_table(value_embed_w, row_embed_w, col_embed_w, box_embed_w,
                       input_bias, ln_gamma, ln_beta)
    t3 = t2d.reshape(SEQ, VPAD, H)
    return _gather(values.astype(jnp.int32), t3)
